# baseline (device time: 84419 ns/iter reference)
import jax
import jax.numpy as jnp
from jax import lax
from jax.experimental import pallas as pl
from jax.experimental.pallas import tpu as pltpu

N_DEV = 32
B, SQ, SKV, HQ, HQ_LOC, DH = 2, 256, 256, 128, 4, 64
D_MODEL = 512
PM = HQ * DH
C_LOC = HQ_LOC * DH
N_CHUNKS = 32
CHUNK_ROWS = (B * SQ) // N_CHUNKS
S_BLK = 32
N_SBLK = SKV // S_BLK


def kernel(x, Wq, K_ext, V_ext, Wo):
    k_pm = jnp.transpose(K_ext, (0, 1, 3, 2)).reshape(B, SKV, PM)
    v_pm = jnp.transpose(V_ext, (0, 1, 3, 2)).reshape(B, SKV, PM)

    def body(x_ref, wq_ref, k_hbm, v_hbm, wo_ref, out_ref,
             acc_ref, comm1_ref, comm2_ref, kbuf, vbuf, kc_ref, vc_ref,
             ksems, vsems, send1, recv1, send2, recv2):
        me = lax.axis_index("i")

        def kv_copy(hbm, buf, slot, i, sem):
            return pltpu.make_async_copy(
                hbm.at[:, pl.ds(i * S_BLK, S_BLK), :], buf.at[slot],
                sem.at[slot])

        for slot in range(2):
            kv_copy(k_hbm, kbuf, slot, slot, ksems).start()
            kv_copy(v_hbm, vbuf, slot, slot, vsems).start()

        barrier = pltpu.get_barrier_semaphore()
        for o in range(N_DEV - 1):
            peer = lax.rem(me + 1 + o, N_DEV)
            pl.semaphore_signal(
                barrier, inc=1, device_id=(peer,),
                device_id_type=pl.DeviceIdType.MESH,
            )
        pl.semaphore_wait(barrier, N_DEV - 1)

        pm_i = lax.broadcasted_iota(jnp.int32, (PM, C_LOC), 0)
        c_i = lax.broadcasted_iota(jnp.int32, (PM, C_LOC), 1)
        ematch = (lax.rem(pm_i, HQ) == HQ_LOC * me + c_i // DH) & (
            pm_i // HQ == lax.rem(c_i, DH))
        e_sel = ematch.astype(jnp.bfloat16)

        for i in range(N_SBLK):
            slot = i % 2
            kv_copy(k_hbm, kbuf, slot, i, ksems).wait()
            kv_copy(v_hbm, vbuf, slot, i, vsems).wait()
            for b in range(B):
                kc_ref[b, i * S_BLK:(i + 1) * S_BLK, :] = jnp.dot(
                    kbuf[slot, b].astype(jnp.bfloat16), e_sel,
                    preferred_element_type=jnp.float32).astype(jnp.bfloat16)
                vc_ref[b, i * S_BLK:(i + 1) * S_BLK, :] = jnp.dot(
                    vbuf[slot, b].astype(jnp.bfloat16), e_sel,
                    preferred_element_type=jnp.float32).astype(jnp.bfloat16)
            if i + 2 < N_SBLK:
                kv_copy(k_hbm, kbuf, slot, i + 2, ksems).start()
                kv_copy(v_hbm, vbuf, slot, i + 2, vsems).start()

        xb = x_ref[...].reshape(B * SQ, D_MODEL).astype(jnp.bfloat16)
        wq = wq_ref[...].astype(jnp.bfloat16)
        q = jnp.dot(xb, wq, preferred_element_type=jnp.float32)

        row_blk = lax.broadcasted_iota(jnp.int32, (SQ, SKV), 0) // 64
        col_blk = lax.broadcasted_iota(jnp.int32, (SQ, SKV), 1) // 64
        mask = col_blk <= row_blk

        ctx_rows = []
        for b in range(B):
            parts = []
            for h in range(HQ_LOC):
                qbh = q[b * SQ:(b + 1) * SQ, h * DH:(h + 1) * DH]
                qbh = qbh.astype(jnp.bfloat16)
                kbh = kc_ref[b, :, h * DH:(h + 1) * DH]
                s = lax.dot_general(
                    qbh, kbh, (((1,), (1,)), ((), ())),
                    preferred_element_type=jnp.float32,
                ) * 0.125
                s = jnp.where(mask, s, -1e9)
                m = jnp.max(s, axis=-1, keepdims=True)
                w = jnp.exp(s - m)
                w = w / jnp.sum(w, axis=-1, keepdims=True)
                vbh = vc_ref[b, :, h * DH:(h + 1) * DH]
                parts.append(jnp.dot(w.astype(jnp.bfloat16), vbh,
                                     preferred_element_type=jnp.float32))
            ctx_rows.append(jnp.concatenate(parts, axis=1))
        ctx = jnp.concatenate(ctx_rows, axis=0).astype(jnp.bfloat16)
        wo = wo_ref[...].astype(jnp.bfloat16)
        partial = jnp.dot(ctx, wo, preferred_element_type=jnp.float32)
        acc_ref[...] = partial.astype(jnp.bfloat16).reshape(
            N_CHUNKS, CHUNK_ROWS, D_MODEL)

        rdmas1 = []
        for o in range(N_DEV - 1):
            j = lax.rem(me + 1 + o, N_DEV)
            r = pltpu.make_async_remote_copy(
                src_ref=acc_ref.at[j],
                dst_ref=comm1_ref.at[N_DEV - 2 - o],
                send_sem=send1.at[o],
                recv_sem=recv1.at[N_DEV - 2 - o],
                device_id=(j,),
                device_id_type=pl.DeviceIdType.MESH,
            )
            r.start()
            rdmas1.append(r)
        for r in rdmas1:
            r.wait_recv()
        reduced = acc_ref[me].astype(jnp.float32) + jnp.sum(
            comm1_ref[...].astype(jnp.float32), axis=0)
        acc_ref[me] = reduced.astype(jnp.bfloat16)
        out_ref[me] = reduced

        rdmas2 = []
        for o in range(N_DEV - 1):
            j = lax.rem(me + 1 + o, N_DEV)
            r = pltpu.make_async_remote_copy(
                src_ref=acc_ref.at[me],
                dst_ref=comm2_ref.at[N_DEV - 2 - o],
                send_sem=send2.at[o],
                recv_sem=recv2.at[N_DEV - 2 - o],
                device_id=(j,),
                device_id_type=pl.DeviceIdType.MESH,
            )
            r.start()
            rdmas2.append(r)
        for t in range(N_DEV - 1):
            rdmas2[N_DEV - 2 - t].wait_recv()
            s_idx = lax.rem(me + 1 + t, N_DEV)
            out_ref[s_idx] = comm2_ref[t].astype(jnp.float32)

        for r in rdmas1:
            r.wait_send()
        for r in rdmas2:
            r.wait_send()

    out = pl.pallas_call(
        body,
        out_shape=jax.ShapeDtypeStruct((N_CHUNKS, CHUNK_ROWS, D_MODEL),
                                       jnp.float32),
        in_specs=[
            pl.BlockSpec(memory_space=pltpu.VMEM),
            pl.BlockSpec(memory_space=pltpu.VMEM),
            pl.BlockSpec(memory_space=pl.ANY),
            pl.BlockSpec(memory_space=pl.ANY),
            pl.BlockSpec(memory_space=pltpu.VMEM),
        ],
        out_specs=pl.BlockSpec(memory_space=pltpu.VMEM),
        scratch_shapes=[
            pltpu.VMEM((N_CHUNKS, CHUNK_ROWS, D_MODEL), jnp.bfloat16),
            pltpu.VMEM((N_DEV - 1, CHUNK_ROWS, D_MODEL), jnp.bfloat16),
            pltpu.VMEM((N_DEV - 1, CHUNK_ROWS, D_MODEL), jnp.bfloat16),
            pltpu.VMEM((2, B, S_BLK, PM), jnp.float32),
            pltpu.VMEM((2, B, S_BLK, PM), jnp.float32),
            pltpu.VMEM((B, SKV, C_LOC), jnp.bfloat16),
            pltpu.VMEM((B, SKV, C_LOC), jnp.bfloat16),
            pltpu.SemaphoreType.DMA((2,)),
            pltpu.SemaphoreType.DMA((2,)),
            pltpu.SemaphoreType.DMA((N_DEV - 1,)),
            pltpu.SemaphoreType.DMA((N_DEV - 1,)),
            pltpu.SemaphoreType.DMA((N_DEV - 1,)),
            pltpu.SemaphoreType.DMA((N_DEV - 1,)),
        ],
        compiler_params=pltpu.CompilerParams(collective_id=0),
    )(x, Wq, k_pm, v_pm, Wo)
    return out.reshape(B, SQ, D_MODEL)


# device time: 40623 ns/iter; 2.0781x vs baseline; 2.0781x over previous
import jax
import jax.numpy as jnp
from jax import lax
from jax.experimental import pallas as pl
from jax.experimental.pallas import tpu as pltpu

N_DEV = 32
B, SQ, SKV, HQ, HQ_LOC, DH = 2, 256, 256, 128, 4, 64
D_MODEL = 512
C_LOC = HQ_LOC * DH
N_CHUNKS = 32
CHUNK_ROWS = (B * SQ) // N_CHUNKS
S_BLK = 32
R_BLK = S_BLK * DH
N_SBLK = SKV // S_BLK


def kernel(x, Wq, K_ext, V_ext, Wo):
    k_r = jnp.transpose(K_ext, (0, 1, 3, 2)).reshape(B, SKV * DH, HQ)
    v_r = jnp.transpose(V_ext, (0, 1, 3, 2)).reshape(B, SKV * DH, HQ)

    def body(x_hbm, wq_hbm, k_hbm, v_hbm, wo_hbm, out_hbm,
             acc_ref, comm1_ref, comm2_ref, kbuf, vbuf, kc_ref, vc_ref,
             x_ref, wq_ref, wo_ref, out_ref,
             ksems, vsems, in_sems, out_sem, send1, recv1, send2, recv2):
        me = lax.axis_index("i")

        def kv_copy(hbm, buf, slot, i, sem):
            return pltpu.make_async_copy(
                hbm.at[:, pl.ds(i * R_BLK, R_BLK), :], buf.at[slot],
                sem.at[slot])

        in_copies = [
            pltpu.make_async_copy(h, v, in_sems.at[n])
            for n, (h, v) in enumerate(
                [(x_hbm, x_ref), (wq_hbm, wq_ref), (wo_hbm, wo_ref)])
        ]
        for c in in_copies:
            c.start()
        for slot in range(2):
            kv_copy(k_hbm, kbuf, slot, slot, ksems).start()
            kv_copy(v_hbm, vbuf, slot, slot, vsems).start()

        barrier = pltpu.get_barrier_semaphore()
        for o in range(N_DEV - 1):
            peer = lax.rem(me + 1 + o, N_DEV)
            pl.semaphore_signal(
                barrier, inc=1, device_id=(peer,),
                device_id_type=pl.DeviceIdType.MESH,
            )

        h_i = lax.broadcasted_iota(jnp.int32, (HQ, C_LOC), 0)
        c_i = lax.broadcasted_iota(jnp.int32, (HQ, C_LOC), 1)
        s_sel = (h_i == HQ_LOC * me + c_i // DH).astype(jnp.bfloat16)
        dh_i = lax.broadcasted_iota(jnp.int32, (S_BLK, DH, C_LOC), 1)
        cm_i = lax.broadcasted_iota(jnp.int32, (S_BLK, DH, C_LOC), 2)
        dh_mask = dh_i == cm_i % DH

        in_copies[0].wait()
        in_copies[1].wait()
        xb = x_ref[...].reshape(B * SQ, D_MODEL).astype(jnp.bfloat16)
        wq = wq_ref[...].astype(jnp.bfloat16)
        q = jnp.dot(xb, wq, preferred_element_type=jnp.float32)

        def extract(buf, slot, dst, i):
            w = jnp.dot(
                buf[slot].reshape(B * R_BLK, HQ).astype(jnp.bfloat16), s_sel,
                preferred_element_type=jnp.float32,
            ).reshape(B, S_BLK, DH, C_LOC)
            for b in range(B):
                dst[b, i * S_BLK:(i + 1) * S_BLK, :] = jnp.sum(
                    jnp.where(dh_mask, w[b], 0.0), axis=1
                ).astype(jnp.bfloat16)

        for i in range(N_SBLK):
            slot = i % 2
            kv_copy(k_hbm, kbuf, slot, i, ksems).wait()
            kv_copy(v_hbm, vbuf, slot, i, vsems).wait()
            extract(kbuf, slot, kc_ref, i)
            extract(vbuf, slot, vc_ref, i)
            if i + 2 < N_SBLK:
                kv_copy(k_hbm, kbuf, slot, i + 2, ksems).start()
                kv_copy(v_hbm, vbuf, slot, i + 2, vsems).start()

        row_blk = lax.broadcasted_iota(jnp.int32, (SQ, SKV), 0) // 64
        col_blk = lax.broadcasted_iota(jnp.int32, (SQ, SKV), 1) // 64
        mask = col_blk <= row_blk

        ctx_rows = []
        for b in range(B):
            parts = []
            for h in range(HQ_LOC):
                qbh = q[b * SQ:(b + 1) * SQ, h * DH:(h + 1) * DH]
                qbh = qbh.astype(jnp.bfloat16)
                kbh = kc_ref[b, :, h * DH:(h + 1) * DH]
                s = lax.dot_general(
                    qbh, kbh, (((1,), (1,)), ((), ())),
                    preferred_element_type=jnp.float32,
                ) * 0.125
                s = jnp.where(mask, s, -1e9)
                m = jnp.max(s, axis=-1, keepdims=True)
                w = jnp.exp(s - m)
                w = w / jnp.sum(w, axis=-1, keepdims=True)
                vbh = vc_ref[b, :, h * DH:(h + 1) * DH]
                parts.append(jnp.dot(w.astype(jnp.bfloat16), vbh,
                                     preferred_element_type=jnp.float32))
            ctx_rows.append(jnp.concatenate(parts, axis=1))
        ctx = jnp.concatenate(ctx_rows, axis=0).astype(jnp.bfloat16)
        in_copies[2].wait()
        wo = wo_ref[...].astype(jnp.bfloat16)
        partial = jnp.dot(ctx, wo, preferred_element_type=jnp.float32)
        acc_ref[...] = partial.astype(jnp.bfloat16).reshape(
            N_CHUNKS, CHUNK_ROWS, D_MODEL)

        pl.semaphore_wait(barrier, N_DEV - 1)

        rdmas1 = []
        for o in range(N_DEV - 1):
            j = lax.rem(me + 1 + o, N_DEV)
            r = pltpu.make_async_remote_copy(
                src_ref=acc_ref.at[j],
                dst_ref=comm1_ref.at[N_DEV - 2 - o],
                send_sem=send1.at[o],
                recv_sem=recv1.at[N_DEV - 2 - o],
                device_id=(j,),
                device_id_type=pl.DeviceIdType.MESH,
            )
            r.start()
            rdmas1.append(r)
        for r in rdmas1:
            r.wait_recv()
        reduced = acc_ref[me].astype(jnp.float32) + jnp.sum(
            comm1_ref[...].astype(jnp.float32), axis=0)
        acc_ref[me] = reduced.astype(jnp.bfloat16)
        out_ref[me] = reduced

        rdmas2 = []
        for o in range(N_DEV - 1):
            j = lax.rem(me + 1 + o, N_DEV)
            r = pltpu.make_async_remote_copy(
                src_ref=acc_ref.at[me],
                dst_ref=comm2_ref.at[N_DEV - 2 - o],
                send_sem=send2.at[o],
                recv_sem=recv2.at[N_DEV - 2 - o],
                device_id=(j,),
                device_id_type=pl.DeviceIdType.MESH,
            )
            r.start()
            rdmas2.append(r)
        for t in range(N_DEV - 1):
            rdmas2[N_DEV - 2 - t].wait_recv()
            s_idx = lax.rem(me + 1 + t, N_DEV)
            out_ref[s_idx] = comm2_ref[t].astype(jnp.float32)

        out_copy = pltpu.make_async_copy(out_ref, out_hbm, out_sem)
        out_copy.start()
        for r in rdmas1:
            r.wait_send()
        for r in rdmas2:
            r.wait_send()
        out_copy.wait()

    out = pl.pallas_call(
        body,
        out_shape=jax.ShapeDtypeStruct((N_CHUNKS, CHUNK_ROWS, D_MODEL),
                                       jnp.float32),
        in_specs=[
            pl.BlockSpec(memory_space=pltpu.MemorySpace.HBM),
            pl.BlockSpec(memory_space=pltpu.MemorySpace.HBM),
            pl.BlockSpec(memory_space=pltpu.MemorySpace.HBM),
            pl.BlockSpec(memory_space=pltpu.MemorySpace.HBM),
            pl.BlockSpec(memory_space=pltpu.MemorySpace.HBM),
        ],
        out_specs=pl.BlockSpec(memory_space=pltpu.MemorySpace.HBM),
        scratch_shapes=[
            pltpu.VMEM((N_CHUNKS, CHUNK_ROWS, D_MODEL), jnp.bfloat16),
            pltpu.VMEM((N_DEV - 1, CHUNK_ROWS, D_MODEL), jnp.bfloat16),
            pltpu.VMEM((N_DEV - 1, CHUNK_ROWS, D_MODEL), jnp.bfloat16),
            pltpu.VMEM((2, B, R_BLK, HQ), jnp.float32),
            pltpu.VMEM((2, B, R_BLK, HQ), jnp.float32),
            pltpu.VMEM((B, SKV, C_LOC), jnp.bfloat16),
            pltpu.VMEM((B, SKV, C_LOC), jnp.bfloat16),
            pltpu.VMEM((B, SQ, D_MODEL), jnp.float32),
            pltpu.VMEM((D_MODEL, C_LOC), jnp.float32),
            pltpu.VMEM((C_LOC, D_MODEL), jnp.float32),
            pltpu.VMEM((N_CHUNKS, CHUNK_ROWS, D_MODEL), jnp.float32),
            pltpu.SemaphoreType.DMA((2,)),
            pltpu.SemaphoreType.DMA((2,)),
            pltpu.SemaphoreType.DMA((3,)),
            pltpu.SemaphoreType.DMA,
            pltpu.SemaphoreType.DMA((N_DEV - 1,)),
            pltpu.SemaphoreType.DMA((N_DEV - 1,)),
            pltpu.SemaphoreType.DMA((N_DEV - 1,)),
            pltpu.SemaphoreType.DMA((N_DEV - 1,)),
        ],
        compiler_params=pltpu.CompilerParams(collective_id=0),
    )(*(pltpu.with_memory_space_constraint(a, pltpu.MemorySpace.HBM)
        for a in (x, Wq, k_r, v_r, Wo)))
    return out.reshape(B, SQ, D_MODEL)


# device time: 39498 ns/iter; 2.1373x vs baseline; 1.0285x over previous
import jax
import jax.numpy as jnp
from jax import lax
from jax.experimental import pallas as pl
from jax.experimental.pallas import tpu as pltpu

N_DEV = 32
B, SQ, SKV, HQ, HQ_LOC, DH = 2, 256, 256, 128, 4, 64
D_MODEL = 512
C_LOC = HQ_LOC * DH
N_CHUNKS = 32
CHUNK_ROWS = (B * SQ) // N_CHUNKS
S_BLK = 64
R_BLK = S_BLK * DH
N_SBLK = SKV // S_BLK


def kernel(x, Wq, K_ext, V_ext, Wo):
    k_r = jnp.transpose(K_ext, (0, 1, 3, 2)).reshape(B, SKV * DH, HQ)
    v_r = jnp.transpose(V_ext, (0, 1, 3, 2)).reshape(B, SKV * DH, HQ)

    def body(x_hbm, wq_hbm, k_hbm, v_hbm, wo_hbm, out_hbm,
             acc_ref, comm1_ref, comm2_ref, kbuf, vbuf, kc_ref, vc_ref,
             x_ref, wq_ref, wo_ref, out_ref,
             ksems, vsems, in_sems, out_sem, send1, recv1, send2, recv2):
        me = lax.axis_index("i")

        def kv_copy(hbm, buf, slot, i, sem):
            return pltpu.make_async_copy(
                hbm.at[:, pl.ds(i * R_BLK, R_BLK), :], buf.at[slot],
                sem.at[slot])

        in_copies = [
            pltpu.make_async_copy(h, v, in_sems.at[n])
            for n, (h, v) in enumerate(
                [(x_hbm, x_ref), (wq_hbm, wq_ref), (wo_hbm, wo_ref)])
        ]
        for c in in_copies:
            c.start()
        for slot in range(2):
            kv_copy(k_hbm, kbuf, slot, slot, ksems).start()
            kv_copy(v_hbm, vbuf, slot, slot, vsems).start()

        barrier = pltpu.get_barrier_semaphore()
        for o in range(N_DEV - 1):
            peer = lax.rem(me + 1 + o, N_DEV)
            pl.semaphore_signal(
                barrier, inc=1, device_id=(peer,),
                device_id_type=pl.DeviceIdType.MESH,
            )

        h_i = lax.broadcasted_iota(jnp.int32, (HQ, C_LOC), 0)
        c_i = lax.broadcasted_iota(jnp.int32, (HQ, C_LOC), 1)
        s_sel = (h_i == HQ_LOC * me + c_i // DH).astype(jnp.bfloat16)
        dh_i = lax.broadcasted_iota(jnp.int32, (S_BLK, DH, C_LOC), 1)
        cm_i = lax.broadcasted_iota(jnp.int32, (S_BLK, DH, C_LOC), 2)
        dh_mask = dh_i == cm_i % DH

        in_copies[0].wait()
        in_copies[1].wait()
        xb = x_ref[...].reshape(B * SQ, D_MODEL).astype(jnp.bfloat16)
        wq = wq_ref[...].astype(jnp.bfloat16)
        q = jnp.dot(xb, wq, preferred_element_type=jnp.float32)

        def extract(buf, slot, dst, i):
            w = jnp.dot(
                buf[slot].reshape(B * R_BLK, HQ).astype(jnp.bfloat16), s_sel,
                preferred_element_type=jnp.float32,
            ).reshape(B, S_BLK, DH, C_LOC)
            for b in range(B):
                dst[b, i * S_BLK:(i + 1) * S_BLK, :] = jnp.sum(
                    jnp.where(dh_mask, w[b], 0.0), axis=1
                ).astype(jnp.bfloat16)

        for i in range(N_SBLK):
            slot = i % 2
            kv_copy(k_hbm, kbuf, slot, i, ksems).wait()
            kv_copy(v_hbm, vbuf, slot, i, vsems).wait()
            extract(kbuf, slot, kc_ref, i)
            extract(vbuf, slot, vc_ref, i)
            if i + 2 < N_SBLK:
                kv_copy(k_hbm, kbuf, slot, i + 2, ksems).start()
                kv_copy(v_hbm, vbuf, slot, i + 2, vsems).start()

        row_blk = lax.broadcasted_iota(jnp.int32, (SQ, SKV), 0) // 64
        col_blk = lax.broadcasted_iota(jnp.int32, (SQ, SKV), 1) // 64
        mask = col_blk <= row_blk

        ctx_rows = []
        for b in range(B):
            parts = []
            for h in range(HQ_LOC):
                qbh = q[b * SQ:(b + 1) * SQ, h * DH:(h + 1) * DH]
                qbh = qbh.astype(jnp.bfloat16)
                kbh = kc_ref[b, :, h * DH:(h + 1) * DH]
                s = lax.dot_general(
                    qbh, kbh, (((1,), (1,)), ((), ())),
                    preferred_element_type=jnp.float32,
                ) * 0.125
                s = jnp.where(mask, s, -1e9)
                m = jnp.max(s, axis=-1, keepdims=True)
                w = jnp.exp(s - m)
                w = w / jnp.sum(w, axis=-1, keepdims=True)
                vbh = vc_ref[b, :, h * DH:(h + 1) * DH]
                parts.append(jnp.dot(w.astype(jnp.bfloat16), vbh,
                                     preferred_element_type=jnp.float32))
            ctx_rows.append(jnp.concatenate(parts, axis=1))
        ctx = jnp.concatenate(ctx_rows, axis=0).astype(jnp.bfloat16)
        in_copies[2].wait()
        wo = wo_ref[...].astype(jnp.bfloat16)
        partial = jnp.dot(ctx, wo, preferred_element_type=jnp.float32)
        acc_ref[...] = partial.astype(jnp.bfloat16).reshape(
            N_CHUNKS, CHUNK_ROWS, D_MODEL)

        pl.semaphore_wait(barrier, N_DEV - 1)

        rdmas1 = []
        for o in range(N_DEV - 1):
            j = lax.rem(me + 1 + o, N_DEV)
            r = pltpu.make_async_remote_copy(
                src_ref=acc_ref.at[j],
                dst_ref=comm1_ref.at[N_DEV - 2 - o],
                send_sem=send1.at[o],
                recv_sem=recv1.at[N_DEV - 2 - o],
                device_id=(j,),
                device_id_type=pl.DeviceIdType.MESH,
            )
            r.start()
            rdmas1.append(r)
        for r in rdmas1:
            r.wait_recv()
        reduced = acc_ref[me].astype(jnp.float32) + jnp.sum(
            comm1_ref[...].astype(jnp.float32), axis=0)
        acc_ref[me] = reduced.astype(jnp.bfloat16)
        out_ref[me] = reduced

        rdmas2 = []
        for o in range(N_DEV - 1):
            j = lax.rem(me + 1 + o, N_DEV)
            r = pltpu.make_async_remote_copy(
                src_ref=acc_ref.at[me],
                dst_ref=comm2_ref.at[N_DEV - 2 - o],
                send_sem=send2.at[o],
                recv_sem=recv2.at[N_DEV - 2 - o],
                device_id=(j,),
                device_id_type=pl.DeviceIdType.MESH,
            )
            r.start()
            rdmas2.append(r)
        for t in range(N_DEV - 1):
            rdmas2[N_DEV - 2 - t].wait_recv()
            s_idx = lax.rem(me + 1 + t, N_DEV)
            out_ref[s_idx] = comm2_ref[t].astype(jnp.float32)

        out_copy = pltpu.make_async_copy(out_ref, out_hbm, out_sem)
        out_copy.start()
        for r in rdmas1:
            r.wait_send()
        for r in rdmas2:
            r.wait_send()
        out_copy.wait()

    out = pl.pallas_call(
        body,
        out_shape=jax.ShapeDtypeStruct((N_CHUNKS, CHUNK_ROWS, D_MODEL),
                                       jnp.float32),
        in_specs=[
            pl.BlockSpec(memory_space=pltpu.MemorySpace.HBM),
            pl.BlockSpec(memory_space=pltpu.MemorySpace.HBM),
            pl.BlockSpec(memory_space=pltpu.MemorySpace.HBM),
            pl.BlockSpec(memory_space=pltpu.MemorySpace.HBM),
            pl.BlockSpec(memory_space=pltpu.MemorySpace.HBM),
        ],
        out_specs=pl.BlockSpec(memory_space=pltpu.MemorySpace.HBM),
        scratch_shapes=[
            pltpu.VMEM((N_CHUNKS, CHUNK_ROWS, D_MODEL), jnp.bfloat16),
            pltpu.VMEM((N_DEV - 1, CHUNK_ROWS, D_MODEL), jnp.bfloat16),
            pltpu.VMEM((N_DEV - 1, CHUNK_ROWS, D_MODEL), jnp.bfloat16),
            pltpu.VMEM((2, B, R_BLK, HQ), jnp.float32),
            pltpu.VMEM((2, B, R_BLK, HQ), jnp.float32),
            pltpu.VMEM((B, SKV, C_LOC), jnp.bfloat16),
            pltpu.VMEM((B, SKV, C_LOC), jnp.bfloat16),
            pltpu.VMEM((B, SQ, D_MODEL), jnp.float32),
            pltpu.VMEM((D_MODEL, C_LOC), jnp.float32),
            pltpu.VMEM((C_LOC, D_MODEL), jnp.float32),
            pltpu.VMEM((N_CHUNKS, CHUNK_ROWS, D_MODEL), jnp.float32),
            pltpu.SemaphoreType.DMA((2,)),
            pltpu.SemaphoreType.DMA((2,)),
            pltpu.SemaphoreType.DMA((3,)),
            pltpu.SemaphoreType.DMA,
            pltpu.SemaphoreType.DMA((N_DEV - 1,)),
            pltpu.SemaphoreType.DMA((N_DEV - 1,)),
            pltpu.SemaphoreType.DMA((N_DEV - 1,)),
            pltpu.SemaphoreType.DMA((N_DEV - 1,)),
        ],
        compiler_params=pltpu.CompilerParams(collective_id=0),
    )(*(pltpu.with_memory_space_constraint(a, pltpu.MemorySpace.HBM)
        for a in (x, Wq, k_r, v_r, Wo)))
    return out.reshape(B, SQ, D_MODEL)
